# baseline (speedup 1.0000x reference)
import jax
import jax.numpy as jnp
from jax import lax
from jax.experimental import pallas as pl
from jax.experimental.pallas import tpu as pltpu
from jax.experimental.pallas import tpu_sc as plsc

_B = 4
_S = 8192
_H = 768
_EPS = 1e-12
_NC = 2
_NS = 16
_NW = _NC * _NS
_SPW = _S // _NW
_K = 64
_NPC = _SPW // _K
_L = 16
_G = _K // _L


def _nr_rsqrt(v):
    i = lax.bitcast_convert_type(v, jnp.int32)
    y = lax.bitcast_convert_type(
        jnp.int32(0x5F3759DF) - lax.shift_right_arithmetic(i, 1), jnp.float32)
    for _ in range(3):
        y = y * (1.5 - 0.5 * v * y * y)
    return y


def _body(ids_hbm, tok_hbm, pos_hbm, gamma_hbm, beta_hbm, out_hbm,
          idx_v, pos_v, tok_v, gam_v, bet_v, sem):
    cid = lax.axis_index("c")
    sid = lax.axis_index("s")
    wid = sid * _NC + cid
    s_base = wid * _SPW

    pltpu.sync_copy(gamma_hbm, gam_v)
    pltpu.sync_copy(beta_hbm, bet_v)

    iota = lax.iota(jnp.int32, _L)
    rows = [iota + g * _L for g in range(_G)]
    zero = jnp.zeros((_L,), jnp.float32)

    def round_body(r, _):
        pc = lax.shift_right_logical(r, 2)
        b = lax.bitwise_and(r, 3)
        s0 = s_base + pc * _K
        off = b * _S + s0

        @pl.when(b == 0)
        def _load_pos():
            pltpu.sync_copy(pos_hbm.at[pl.ds(s0, _K)], pos_v)

        pltpu.sync_copy(ids_hbm.at[pl.ds(off, _K)], idx_v)
        pltpu.async_copy(tok_hbm.at[idx_v], tok_v, sem).wait()

        def p1(c, carry):
            s1 = list(carry[:_G])
            s2 = list(carry[_G:])
            colv = jnp.full((_L,), c, dtype=jnp.int32)
            for g in range(_G):
                vt = plsc.load_gather(tok_v, [rows[g], colv])
                vp = plsc.load_gather(pos_v, [rows[g], colv])
                v = vt + vp
                plsc.store_scatter(tok_v, [rows[g], colv], v)
                s1[g] = s1[g] + v
                s2[g] = s2[g] + v * v
            return tuple(s1) + tuple(s2)

        carry = lax.fori_loop(0, _H, p1, (zero,) * (2 * _G))
        inv = jnp.float32(1.0 / _H)
        rg = []
        nmrg = []
        for g in range(_G):
            m = carry[g] * inv
            var = carry[_G + g] * inv - m * m
            rr = _nr_rsqrt(var + jnp.float32(_EPS))
            rg.append(rr)
            nmrg.append(-(m * rr))

        def p2(c, acc):
            colv = jnp.full((_L,), c, dtype=jnp.int32)
            gc = plsc.load_gather(gam_v, [colv])
            bc = plsc.load_gather(bet_v, [colv])
            for g in range(_G):
                x = plsc.load_gather(tok_v, [rows[g], colv])
                y = (x * rg[g] + nmrg[g]) * gc + bc
                plsc.store_scatter(tok_v, [rows[g], colv], y)
            return acc

        lax.fori_loop(0, _H, p2, 0)
        pltpu.sync_copy(tok_v, out_hbm.at[pl.ds(off, _K)])
        return 0

    lax.fori_loop(0, _NPC * _B, round_body, 0)


_mesh = plsc.VectorSubcoreMesh(
    core_axis_name="c", subcore_axis_name="s", num_cores=_NC, num_subcores=_NS)

_embed_ln = pl.kernel(
    _body,
    out_type=jax.ShapeDtypeStruct((_B * _S, _H), jnp.float32),
    mesh=_mesh,
    scratch_types=[
        pltpu.VMEM((_K,), jnp.int32),
        pltpu.VMEM((_K, _H), jnp.float32),
        pltpu.VMEM((_K, _H), jnp.float32),
        pltpu.VMEM((_H,), jnp.float32),
        pltpu.VMEM((_H,), jnp.float32),
        pltpu.SemaphoreType.DMA,
    ],
    compiler_params=pltpu.CompilerParams(
        use_tc_tiling_on_sc=False, needs_layout_passes=False),
)


def kernel(input_ids, tok_table, pos_table, gamma, beta):
    ids = input_ids.reshape(-1).astype(jnp.int32)
    out = _embed_ln(ids, tok_table, pos_table, gamma, beta)
    return out.reshape(_B, _S, _H)


# speedup vs baseline: 1.7810x; 1.7810x over previous
import jax
import jax.numpy as jnp
from jax import lax
from jax.experimental import pallas as pl
from jax.experimental.pallas import tpu as pltpu
from jax.experimental.pallas import tpu_sc as plsc

_B = 4
_S = 8192
_H = 768
_EPS = 1e-12
_NC = 2
_NS = 16
_NW = _NC * _NS
_SPW = _S // _NW
_K = 32
_NPC = _SPW // _K
_L = 16
_G = _K // _L
_NROUND = _NPC * _B
_UNROLL = 8


def _nr_rsqrt(v):
    i = lax.bitcast_convert_type(v, jnp.int32)
    y = lax.bitcast_convert_type(
        jnp.int32(0x5F3759DF) - lax.shift_right_arithmetic(i, 1), jnp.float32)
    for _ in range(3):
        y = y * (1.5 - 0.5 * v * y * y)
    return y


def _body(ids_hbm, tok_hbm, pos_hbm, gamma_hbm, beta_hbm, out_hbm,
          idx_all, pos_v, tok_a, tok_b, gam_v, bet_v,
          gsem_a, gsem_b, osem_a, osem_b):
    cid = lax.axis_index("c")
    sid = lax.axis_index("s")
    wid = sid * _NC + cid
    s_base = wid * _SPW

    pltpu.sync_copy(gamma_hbm, gam_v)
    pltpu.sync_copy(beta_hbm, bet_v)
    for b in range(_B):
        pltpu.sync_copy(ids_hbm.at[pl.ds(b * _S + s_base, _SPW)],
                        idx_all.at[pl.ds(b * _SPW, _SPW)])

    iota = lax.iota(jnp.int32, _L)
    rows = [iota + g * _L for g in range(_G)]
    zf = jnp.zeros((_L,), jnp.float32)
    zi = jnp.zeros((_L,), jnp.int32)

    def idx_slice(r):
        pc = lax.shift_right_logical(r, 2)
        b = lax.bitwise_and(r, 3)
        return idx_all.at[pl.ds(b * _SPW + pc * _K, _K)]

    def compute_chunk(tok_buf):
        def p1(c, carry):
            colv = carry[0]
            s = list(carry[1:])
            for g in range(_G):
                vt = plsc.load_gather(tok_buf, [rows[g], colv])
                vp = plsc.load_gather(pos_v, [rows[g], colv])
                v = vt + vp
                plsc.store_scatter(tok_buf, [rows[g], colv], v)
                s[g] = s[g] + v
                s[_G + g] = s[_G + g] + v * v
            return (colv + 1,) + tuple(s)

        res = plsc.parallel_loop(
            0, _H, 1, unroll=_UNROLL, carry=(zi,) + (zf,) * (2 * _G))(p1)
        inv = jnp.float32(1.0 / _H)
        rg = []
        nmrg = []
        for g in range(_G):
            m = res[1 + g] * inv
            var = res[1 + _G + g] * inv - m * m
            rr = _nr_rsqrt(var + jnp.float32(_EPS))
            rg.append(rr)
            nmrg.append(-(m * rr))

        def p2(c, colv):
            gc = plsc.load_gather(gam_v, [colv])
            bc = plsc.load_gather(bet_v, [colv])
            for g in range(_G):
                x = plsc.load_gather(tok_buf, [rows[g], colv])
                y = (x * rg[g] + nmrg[g]) * gc + bc
                plsc.store_scatter(tok_buf, [rows[g], colv], y)
            return colv + 1

        plsc.parallel_loop(0, _H, 1, unroll=_UNROLL, carry=zi)(p2)

    def do_round(r, tok_cur, gsem_cur, osem_cur, tok_nxt, gsem_nxt, osem_nxt):
        pc = lax.shift_right_logical(r, 2)
        b = lax.bitwise_and(r, 3)
        s0 = s_base + pc * _K
        off = b * _S + s0
        out_sl = out_hbm.at[pl.ds(off, _K)]

        @pl.when(b == 0)
        def _():
            pltpu.sync_copy(pos_hbm.at[pl.ds(s0, _K)], pos_v)

        pltpu.make_async_copy(tok_hbm.at[idx_slice(r)], tok_cur, gsem_cur).wait()

        @pl.when(r < _NROUND - 1)
        def _():
            @pl.when(r >= 1)
            def _():
                pltpu.make_async_copy(tok_nxt, out_sl, osem_nxt).wait()
            pltpu.async_copy(tok_hbm.at[idx_slice(r + 1)], tok_nxt, gsem_nxt)

        compute_chunk(tok_cur)
        pltpu.async_copy(tok_cur, out_sl, osem_cur)

    pltpu.async_copy(tok_hbm.at[idx_slice(jnp.int32(0))], tok_a, gsem_a)

    def pair(jj, _):
        r0 = jj * 2
        do_round(r0, tok_a, gsem_a, osem_a, tok_b, gsem_b, osem_b)
        do_round(r0 + 1, tok_b, gsem_b, osem_b, tok_a, gsem_a, osem_a)
        return 0

    lax.fori_loop(0, _NROUND // 2, pair, 0)

    pltpu.make_async_copy(tok_a, out_hbm.at[pl.ds(s_base, _K)], osem_a).wait()
    pltpu.make_async_copy(tok_b, out_hbm.at[pl.ds(s_base, _K)], osem_b).wait()


_mesh = plsc.VectorSubcoreMesh(
    core_axis_name="c", subcore_axis_name="s", num_cores=_NC, num_subcores=_NS)

_embed_ln = pl.kernel(
    _body,
    out_type=jax.ShapeDtypeStruct((_B * _S, _H), jnp.float32),
    mesh=_mesh,
    scratch_types=[
        pltpu.VMEM((_B * _SPW,), jnp.int32),
        pltpu.VMEM((_K, _H), jnp.float32),
        pltpu.VMEM((_K, _H), jnp.float32),
        pltpu.VMEM((_K, _H), jnp.float32),
        pltpu.VMEM((_H,), jnp.float32),
        pltpu.VMEM((_H,), jnp.float32),
        pltpu.SemaphoreType.DMA,
        pltpu.SemaphoreType.DMA,
        pltpu.SemaphoreType.DMA,
        pltpu.SemaphoreType.DMA,
    ],
    compiler_params=pltpu.CompilerParams(
        use_tc_tiling_on_sc=False, needs_layout_passes=False),
)


def kernel(input_ids, tok_table, pos_table, gamma, beta):
    ids = input_ids.reshape(-1).astype(jnp.int32)
    out = _embed_ln(ids, tok_table, pos_table, gamma, beta)
    return out.reshape(_B, _S, _H)


# speedup vs baseline: 6.5327x; 3.6679x over previous
import jax
import jax.numpy as jnp
from jax import lax
from jax.experimental import pallas as pl
from jax.experimental.pallas import tpu as pltpu
from jax.experimental.pallas import tpu_sc as plsc

_B = 4
_S = 8192
_H = 768
_EPS = 1e-12
_NC = 2
_NS = 16
_NW = _NC * _NS
_SPW = _S // _NW
_K = 32
_NPC = _SPW // _K
_L = 16
_G = _K // _L
_NROUND = _NPC * _B
_UNROLL = 8


def _nr_rsqrt(v):
    i = lax.bitcast_convert_type(v, jnp.int32)
    y = lax.bitcast_convert_type(
        jnp.int32(0x5F3759DF) - lax.shift_right_arithmetic(i, 1), jnp.float32)
    for _ in range(3):
        y = y * (1.5 - 0.5 * v * y * y)
    return y


def _tree_sum(vs):
    vs = list(vs)
    while len(vs) > 1:
        nxt = [vs[i] + vs[i + 1] for i in range(0, len(vs) - 1, 2)]
        if len(vs) % 2:
            nxt.append(vs[-1])
        vs = nxt
    return vs[0]


def _body(ids_hbm, tok_hbm, pos_hbm, gamma_hbm, beta_hbm, out_hbm,
          idx_all, pos_v, tok_a, tok_b, gam_v, bet_v, r_st, nmr_st,
          gsem_a, gsem_b, osem_a, osem_b):
    cid = lax.axis_index("c")
    sid = lax.axis_index("s")
    wid = sid * _NC + cid
    s_base = wid * _SPW

    pltpu.sync_copy(gamma_hbm, gam_v)
    pltpu.sync_copy(beta_hbm, bet_v)
    for b in range(_B):
        pltpu.sync_copy(ids_hbm.at[pl.ds(b * _S + s_base, _SPW)],
                        idx_all.at[pl.ds(b * _SPW, _SPW)])

    def idx_slice(r):
        pc = lax.shift_right_logical(r, 2)
        b = lax.bitwise_and(r, 3)
        return idx_all.at[pl.ds(b * _SPW + pc * _K, _K)]

    inv = jnp.float32(1.0 / _H)
    nchunk = _H // _L

    def compute_chunk(tok_buf):
        def p1(r):
            vs = []
            for cc in range(nchunk):
                sl = pl.ds(cc * _L, _L)
                v = tok_buf[r, sl] + pos_v[r, sl]
                tok_buf[r, sl] = v
                vs.append(v)
            s1 = _tree_sum(vs)
            s2 = _tree_sum([v * v for v in vs])
            mv = jnp.full((_L,), jnp.sum(s1)) * inv
            qv = jnp.full((_L,), jnp.sum(s2)) * inv
            rr = _nr_rsqrt(qv - mv * mv + jnp.float32(_EPS))
            r_st[r, :] = rr
            nmr_st[r, :] = -(mv * rr)

        plsc.parallel_loop(0, _K, 1, unroll=2)(p1)

        strip = 8
        for s in range(nchunk // strip):
            gs = [gam_v[pl.ds((s * strip + j) * _L, _L)] for j in range(strip)]
            bs = [bet_v[pl.ds((s * strip + j) * _L, _L)] for j in range(strip)]

            def p2(r, _gs=gs, _bs=bs, _s=s):
                rv = r_st[r, :]
                nv = nmr_st[r, :]
                for j in range(strip):
                    sl = pl.ds((_s * strip + j) * _L, _L)
                    x = tok_buf[r, sl]
                    tok_buf[r, sl] = (x * rv + nv) * _gs[j] + _bs[j]

            plsc.parallel_loop(0, _K, 1, unroll=2)(p2)

    def do_round(r, tok_cur, gsem_cur, osem_cur, tok_nxt, gsem_nxt, osem_nxt):
        pc = lax.shift_right_logical(r, 2)
        b = lax.bitwise_and(r, 3)
        s0 = s_base + pc * _K
        off = b * _S + s0
        out_sl = out_hbm.at[pl.ds(off, _K)]

        @pl.when(b == 0)
        def _():
            pltpu.sync_copy(pos_hbm.at[pl.ds(s0, _K)], pos_v)

        pltpu.make_async_copy(tok_hbm.at[idx_slice(r)], tok_cur, gsem_cur).wait()

        @pl.when(r < _NROUND - 1)
        def _():
            @pl.when(r >= 1)
            def _():
                pltpu.make_async_copy(tok_nxt, out_sl, osem_nxt).wait()
            pltpu.async_copy(tok_hbm.at[idx_slice(r + 1)], tok_nxt, gsem_nxt)

        compute_chunk(tok_cur)
        pltpu.async_copy(tok_cur, out_sl, osem_cur)

    pltpu.async_copy(tok_hbm.at[idx_slice(jnp.int32(0))], tok_a, gsem_a)

    def pair(jj, _):
        r0 = jj * 2
        do_round(r0, tok_a, gsem_a, osem_a, tok_b, gsem_b, osem_b)
        do_round(r0 + 1, tok_b, gsem_b, osem_b, tok_a, gsem_a, osem_a)
        return 0

    lax.fori_loop(0, _NROUND // 2, pair, 0)

    pltpu.make_async_copy(tok_a, out_hbm.at[pl.ds(s_base, _K)], osem_a).wait()
    pltpu.make_async_copy(tok_b, out_hbm.at[pl.ds(s_base, _K)], osem_b).wait()


_mesh = plsc.VectorSubcoreMesh(
    core_axis_name="c", subcore_axis_name="s", num_cores=_NC, num_subcores=_NS)

_embed_ln = pl.kernel(
    _body,
    out_type=jax.ShapeDtypeStruct((_B * _S, _H), jnp.float32),
    mesh=_mesh,
    scratch_types=[
        pltpu.VMEM((_B * _SPW,), jnp.int32),
        pltpu.VMEM((_K, _H), jnp.float32),
        pltpu.VMEM((_K, _H), jnp.float32),
        pltpu.VMEM((_K, _H), jnp.float32),
        pltpu.VMEM((_H,), jnp.float32),
        pltpu.VMEM((_H,), jnp.float32),
        pltpu.VMEM((_K, _L), jnp.float32),
        pltpu.VMEM((_K, _L), jnp.float32),
        pltpu.SemaphoreType.DMA,
        pltpu.SemaphoreType.DMA,
        pltpu.SemaphoreType.DMA,
        pltpu.SemaphoreType.DMA,
    ],
    compiler_params=pltpu.CompilerParams(
        use_tc_tiling_on_sc=False, needs_layout_passes=False),
)


def kernel(input_ids, tok_table, pos_table, gamma, beta):
    ids = input_ids.reshape(-1).astype(jnp.int32)
    out = _embed_ln(ids, tok_table, pos_table, gamma, beta)
    return out.reshape(_B, _S, _H)


# speedup vs baseline: 13.0629x; 1.9996x over previous
import jax
import jax.numpy as jnp
from jax import lax
from jax.experimental import pallas as pl
from jax.experimental.pallas import tpu as pltpu
from jax.experimental.pallas import tpu_sc as plsc

_B = 4
_S = 8192
_H = 768
_EPS = 1e-12
_NC = 2
_NS = 16
_NW = _NC * _NS
_SPW = _S // _NW
_K = 32
_NPC = _SPW // _K
_L = 16
_NROUND = _NPC * _B


def _nr_rsqrt(v):
    i = lax.bitcast_convert_type(v, jnp.int32)
    y = lax.bitcast_convert_type(
        jnp.int32(0x5F3759DF) - lax.shift_right_arithmetic(i, 1), jnp.float32)
    for _ in range(3):
        y = y * (1.5 - 0.5 * v * y * y)
    return y


def _tree_sum(vs):
    vs = list(vs)
    while len(vs) > 1:
        nxt = [vs[i] + vs[i + 1] for i in range(0, len(vs) - 1, 2)]
        if len(vs) % 2:
            nxt.append(vs[-1])
        vs = nxt
    return vs[0]


def _body(ids_hbm, tok_hbm, pos_hbm, gamma_hbm, beta_hbm, out_hbm,
          idx_all, pos_v, tok_a, tok_b, gam_v, bet_v, r_st, nmr_st,
          gsem_a, gsem_b, osem_a, osem_b):
    cid = lax.axis_index("c")
    sid = lax.axis_index("s")
    wid = sid * _NC + cid
    s_base = wid * _SPW

    pltpu.sync_copy(gamma_hbm, gam_v)
    pltpu.sync_copy(beta_hbm, bet_v)
    for b in range(_B):
        pltpu.sync_copy(ids_hbm.at[b, pl.ds(s_base, _SPW)],
                        idx_all.at[pl.ds(b * _SPW, _SPW)])

    def idx_slice(r):
        pc = lax.shift_right_logical(r, 2)
        b = lax.bitwise_and(r, 3)
        return idx_all.at[pl.ds(b * _SPW + pc * _K, _K)]

    inv = jnp.float32(1.0 / _H)
    nchunk = _H // _L

    def compute_chunk(tok_buf):
        def p1(r):
            vs = []
            for cc in range(nchunk):
                sl = pl.ds(cc * _L, _L)
                v = tok_buf[r, sl] + pos_v[r, sl]
                tok_buf[r, sl] = v
                vs.append(v)
            s1 = _tree_sum(vs)
            s2 = _tree_sum([v * v for v in vs])
            mv = jnp.full((_L,), jnp.sum(s1)) * inv
            qv = jnp.full((_L,), jnp.sum(s2)) * inv
            rr = _nr_rsqrt(qv - mv * mv + jnp.float32(_EPS))
            r_st[r, :] = rr
            nmr_st[r, :] = -(mv * rr)

        plsc.parallel_loop(0, _K, 1, unroll=2)(p1)

        strip = 8
        for s in range(nchunk // strip):
            gs = [gam_v[pl.ds((s * strip + j) * _L, _L)] for j in range(strip)]
            bs = [bet_v[pl.ds((s * strip + j) * _L, _L)] for j in range(strip)]

            def p2(r, _gs=gs, _bs=bs, _s=s):
                rv = r_st[r, :]
                nv = nmr_st[r, :]
                for j in range(strip):
                    sl = pl.ds((_s * strip + j) * _L, _L)
                    x = tok_buf[r, sl]
                    tok_buf[r, sl] = (x * rv + nv) * _gs[j] + _bs[j]

            plsc.parallel_loop(0, _K, 1, unroll=2)(p2)

    def do_round(r, tok_cur, gsem_cur, osem_cur, tok_nxt, gsem_nxt, osem_nxt):
        pc = lax.shift_right_logical(r, 2)
        b = lax.bitwise_and(r, 3)
        s0 = s_base + pc * _K
        out_sl = out_hbm.at[b, pl.ds(s0, _K)]

        @pl.when(b == 0)
        def _():
            pltpu.sync_copy(pos_hbm.at[pl.ds(s0, _K)], pos_v)

        pltpu.make_async_copy(tok_hbm.at[idx_slice(r)], tok_cur, gsem_cur).wait()

        @pl.when(r < _NROUND - 1)
        def _():
            @pl.when(r >= 1)
            def _():
                pltpu.make_async_copy(tok_nxt, out_sl, osem_nxt).wait()
            pltpu.async_copy(tok_hbm.at[idx_slice(r + 1)], tok_nxt, gsem_nxt)

        compute_chunk(tok_cur)
        pltpu.async_copy(tok_cur, out_sl, osem_cur)

    pltpu.async_copy(tok_hbm.at[idx_slice(jnp.int32(0))], tok_a, gsem_a)

    def pair(jj, _):
        r0 = jj * 2
        do_round(r0, tok_a, gsem_a, osem_a, tok_b, gsem_b, osem_b)
        do_round(r0 + 1, tok_b, gsem_b, osem_b, tok_a, gsem_a, osem_a)
        return 0

    lax.fori_loop(0, _NROUND // 2, pair, 0)

    pltpu.make_async_copy(tok_a, out_hbm.at[0, pl.ds(s_base, _K)], osem_a).wait()
    pltpu.make_async_copy(tok_b, out_hbm.at[0, pl.ds(s_base, _K)], osem_b).wait()


_mesh = plsc.VectorSubcoreMesh(
    core_axis_name="c", subcore_axis_name="s", num_cores=_NC, num_subcores=_NS)

_embed_ln = pl.kernel(
    _body,
    out_type=jax.ShapeDtypeStruct((_B, _S, _H), jnp.float32),
    mesh=_mesh,
    scratch_types=[
        pltpu.VMEM((_B * _SPW,), jnp.int32),
        pltpu.VMEM((_K, _H), jnp.float32),
        pltpu.VMEM((_K, _H), jnp.float32),
        pltpu.VMEM((_K, _H), jnp.float32),
        pltpu.VMEM((_H,), jnp.float32),
        pltpu.VMEM((_H,), jnp.float32),
        pltpu.VMEM((_K, _L), jnp.float32),
        pltpu.VMEM((_K, _L), jnp.float32),
        pltpu.SemaphoreType.DMA,
        pltpu.SemaphoreType.DMA,
        pltpu.SemaphoreType.DMA,
        pltpu.SemaphoreType.DMA,
    ],
    compiler_params=pltpu.CompilerParams(
        use_tc_tiling_on_sc=True, needs_layout_passes=False),
)


def kernel(input_ids, tok_table, pos_table, gamma, beta):
    return _embed_ln(input_ids.astype(jnp.int32), tok_table, pos_table,
                     gamma, beta)
